# baseline (speedup 1.0000x reference)
import functools

import jax
import jax.numpy as jnp
from jax import lax
from jax.experimental import pallas as pl
from jax.experimental.pallas import tpu as pltpu

NUM_EXPERTS = 8
TOP_K = 2
MODEL_DIM = 768
INTER_DIM = 768
T = 2048
N = T * TOP_K
BM = 128
NPAD = N + NUM_EXPERTS * BM
NWIN = NPAD // BM
_INTERPRET = False


def _routing_body(x_ref, gwt_ref, e1_ref, e2_ref):
    x = x_ref[...]
    logits = lax.dot_general(
        x, gwt_ref[...], (((1,), (0,)), ((), ())),
        preferred_element_type=jnp.float32)
    idx = lax.broadcasted_iota(jnp.int32, logits.shape, 1)
    r1 = jnp.max(logits, axis=1, keepdims=True)
    i1 = jnp.min(jnp.where(logits == r1, idx, NUM_EXPERTS),
                 axis=1, keepdims=True)
    l2 = jnp.where(idx == i1, -jnp.inf, logits)
    r2 = jnp.max(l2, axis=1, keepdims=True)
    i2 = jnp.min(jnp.where(l2 == r2, idx, NUM_EXPERTS),
                 axis=1, keepdims=True)
    e1_ref[...] = i1
    e2_ref[...] = i2


def _routing(x, gwt):
    bt = 256
    return pl.pallas_call(
        _routing_body,
        grid=(T // bt,),
        in_specs=[
            pl.BlockSpec((bt, MODEL_DIM), lambda i: (i, 0)),
            pl.BlockSpec((MODEL_DIM, NUM_EXPERTS), lambda i: (0, 0)),
        ],
        out_specs=[
            pl.BlockSpec((bt, 1), lambda i: (i, 0)),
            pl.BlockSpec((bt, 1), lambda i: (i, 0)),
        ],
        out_shape=[
            jax.ShapeDtypeStruct((T, 1), jnp.int32),
            jax.ShapeDtypeStruct((T, 1), jnp.int32),
        ],
        interpret=_INTERPRET,
    )(x, gwt)


def _dispatch_jnp(x, eids):
    counts = jnp.bincount(eids, length=NUM_EXPERTS)
    padded = (counts + BM - 1) // BM * BM
    pad_start = jnp.concatenate([jnp.zeros((1,), jnp.int32),
                                 jnp.cumsum(padded)[:-1].astype(jnp.int32)])
    cs_excl = jnp.concatenate([jnp.zeros((1,), jnp.int32),
                               jnp.cumsum(counts)[:-1].astype(jnp.int32)])
    order = jnp.argsort(eids, stable=True)
    eids_sorted = eids[order]
    rank = jnp.arange(N, dtype=jnp.int32) - cs_excl[eids_sorted]
    pos_sorted = pad_start[eids_sorted] + rank
    inv = jnp.zeros((N,), jnp.int32).at[order].set(pos_sorted)
    tok = jnp.arange(N, dtype=jnp.int32) % T
    x_sorted = jnp.zeros((NPAD, MODEL_DIM), x.dtype).at[inv].set(x[tok])
    pad_incl = pad_start + padded
    wstarts = jnp.arange(NWIN, dtype=jnp.int32) * BM
    wexp = jnp.minimum(
        jnp.sum(wstarts[:, None] >= pad_incl[None, :], axis=1),
        NUM_EXPERTS - 1).astype(jnp.int32)
    return x_sorted, inv, wexp, pad_start.astype(jnp.int32), counts.astype(jnp.int32)


def _gemm_body(wexp_ref, ps_ref, cnt_ref, xs_ref, u_ref, d_ref, out_ref):
    w = pl.program_id(0)
    e = wexp_ref[w]
    loc = (lax.broadcasted_iota(jnp.int32, (BM, 1), 0)
           + w * BM - ps_ref[e])
    active = loc < cnt_ref[e]
    x = jnp.where(active, xs_ref[...], 0.0).astype(jnp.bfloat16)
    u = lax.dot_general(x, u_ref[0], (((1,), (0,)), ((), ())),
                        preferred_element_type=jnp.float32)
    h = (u * u).astype(jnp.bfloat16)
    out_ref[...] = lax.dot_general(h, d_ref[0], (((1,), (0,)), ((), ())),
                                   preferred_element_type=jnp.float32)


def _grouped_gemm(x_sorted, u, d, wexp, pad_start, counts):
    grid_spec = pltpu.PrefetchScalarGridSpec(
        num_scalar_prefetch=3,
        grid=(NWIN,),
        in_specs=[
            pl.BlockSpec((BM, MODEL_DIM), lambda i, wexp, ps, cnt: (i, 0)),
            pl.BlockSpec((1, MODEL_DIM, INTER_DIM),
                         lambda i, wexp, ps, cnt: (wexp[i], 0, 0)),
            pl.BlockSpec((1, INTER_DIM, MODEL_DIM),
                         lambda i, wexp, ps, cnt: (wexp[i], 0, 0)),
        ],
        out_specs=pl.BlockSpec((BM, MODEL_DIM), lambda i, wexp, ps, cnt: (i, 0)),
    )
    return pl.pallas_call(
        _gemm_body,
        grid_spec=grid_spec,
        out_shape=jax.ShapeDtypeStruct((NPAD, MODEL_DIM), jnp.float32),
        interpret=_INTERPRET,
    )(wexp, pad_start, counts, x_sorted, u, d)


@functools.partial(jax.jit, static_argnames=())
def kernel(hidden_states, gate_w, gate_up_w, down_w):
    orig_shape = hidden_states.shape
    x = hidden_states.reshape(-1, orig_shape[-1])
    u = gate_up_w[:, :, INTER_DIM:].astype(jnp.bfloat16)
    d = down_w.astype(jnp.bfloat16)

    e1, e2 = _routing(x, gate_w.T)
    eids = jnp.concatenate([e1, e2], axis=0).reshape(-1)

    x_sorted, inv, wexp, pad_start, counts = _dispatch_jnp(x, eids)
    rows = _grouped_gemm(x_sorted, u, d, wexp, pad_start, counts)
    out = rows[inv[:T]] + rows[inv[T:]]
    return out.reshape(orig_shape)


# speedup vs baseline: 1.7752x; 1.7752x over previous
import functools

import jax
import jax.numpy as jnp
from jax import lax
from jax.experimental import pallas as pl
from jax.experimental.pallas import tpu as pltpu

NUM_EXPERTS = 8
TOP_K = 2
MODEL_DIM = 768
INTER_DIM = 768
T = 2048
N = T * TOP_K
BM = 128
NPAD = N + NUM_EXPERTS * BM
NWIN = NPAD // BM
_INTERPRET = False


def _routing_body(x_ref, gwt_ref, e1_ref, e2_ref):
    x = x_ref[...]
    logits = lax.dot_general(
        x, gwt_ref[...], (((1,), (0,)), ((), ())),
        preferred_element_type=jnp.float32)
    idx = lax.broadcasted_iota(jnp.int32, logits.shape, 1)
    r1 = jnp.max(logits, axis=1, keepdims=True)
    i1 = jnp.min(jnp.where(logits == r1, idx, NUM_EXPERTS),
                 axis=1, keepdims=True)
    l2 = jnp.where(idx == i1, -jnp.inf, logits)
    r2 = jnp.max(l2, axis=1, keepdims=True)
    i2 = jnp.min(jnp.where(l2 == r2, idx, NUM_EXPERTS),
                 axis=1, keepdims=True)
    e1_ref[...] = i1
    e2_ref[...] = i2


def _routing(x, gwt):
    bt = 256
    return pl.pallas_call(
        _routing_body,
        grid=(T // bt,),
        in_specs=[
            pl.BlockSpec((bt, MODEL_DIM), lambda i: (i, 0)),
            pl.BlockSpec((MODEL_DIM, NUM_EXPERTS), lambda i: (0, 0)),
        ],
        out_specs=[
            pl.BlockSpec((bt, 1), lambda i: (i, 0)),
            pl.BlockSpec((bt, 1), lambda i: (i, 0)),
        ],
        out_shape=[
            jax.ShapeDtypeStruct((T, 1), jnp.int32),
            jax.ShapeDtypeStruct((T, 1), jnp.int32),
        ],
        interpret=_INTERPRET,
    )(x, gwt)


NC = 2
NS = 16
LANES = 16
NW = NC * NS
SLOTS_W = N // NW
CHUNK = 32
NCHUNK = SLOTS_W // CHUNK
TOK_W = T // NW
NWEXP = 48


def _sc_mesh():
    from jax.experimental.pallas import tpu_sc as plsc
    return plsc.VectorSubcoreMesh(core_axis_name="c", subcore_axis_name="s")


def _dispatch_body(eids_hbm, x_hbm,
                   xs_hbm, inv_hbm, wexp_hbm, ps_hbm, cnt_hbm,
                   eids_v, tot_v, pre_v, pos2d, inv_v, meta_v, wexp_v,
                   bufa, bufb, sema, semb):
    from jax.experimental.pallas import tpu_sc as plsc
    wid = lax.axis_index("c") * NS + lax.axis_index("s")
    base = wid * SLOTS_W
    lane = lax.iota(jnp.int32, LANES)
    ones = jnp.ones((LANES,), jnp.int32)

    pltpu.sync_copy(eids_hbm, eids_v)

    tot_v[...] = jnp.zeros((LANES,), jnp.int32)
    pre_v[...] = jnp.zeros((LANES,), jnp.int32)
    klim = wid * (SLOTS_W // LANES)

    def count_body(k, _):
        v = eids_v[pl.ds(k * LANES, LANES)]
        plsc.addupdate_scatter(tot_v, [v], ones)

        @pl.when(k < klim)
        def _():
            plsc.addupdate_scatter(pre_v, [v], ones)
        return 0

    lax.fori_loop(0, N // LANES, count_body, 0)

    tot = tot_v[...]
    pre = pre_v[...]
    pe = jnp.bitwise_and(tot + (BM - 1), -BM)
    incl = plsc.cumsum(pe)
    ps = incl - pe
    myb = ps + pre

    @pl.when(wid == 0)
    def _():
        meta_v[...] = ps
        pltpu.sync_copy(meta_v, ps_hbm)
        meta_v[...] = tot
        pltpu.sync_copy(meta_v, cnt_hbm)
        for k in range(NWEXP // LANES):
            wstart = (lane + k * LANES) * BM
            acc = jnp.zeros((LANES,), jnp.int32)
            for e in range(NUM_EXPERTS):
                incl_e = jnp.sum(jnp.where(lane == e, incl, 0))
                acc = acc + (wstart >= incl_e).astype(jnp.int32)
            wexp_v[pl.ds(k * LANES, LANES)] = jnp.minimum(acc, NUM_EXPERTS - 1)
        pltpu.sync_copy(wexp_v, wexp_hbm)

    for k in range(SLOTS_W // LANES):
        v = eids_v[pl.ds(base + k * LANES, LANES)]
        pos = jnp.zeros((LANES,), jnp.int32)
        for e in range(NUM_EXPERTS):
            m = v == e
            mi = m.astype(jnp.int32)
            r = plsc.cumsum(mi)
            base_e = jnp.sum(jnp.where(lane == e, myb, 0))
            pos = jnp.where(m, base_e + r - 1, pos)
            cnt_e = jnp.sum(mi)
            myb = myb + jnp.where(lane == e, cnt_e, 0)
        pos2d[k // 2, pl.ds((k % 2) * LANES, LANES)] = pos
        inv_v[pl.ds(k * LANES, LANES)] = pos
    pltpu.sync_copy(inv_v, inv_hbm.at[pl.ds(base, SLOTS_W)])

    handles = [None] * NCHUNK
    for c in range(NCHUNK):
        buf, sem = (bufa, sema) if c % 2 == 0 else (bufb, semb)
        if c >= 2:
            handles[c - 2].wait()
        tokbase = lax.rem(base + c * CHUNK, T)
        pltpu.sync_copy(x_hbm.at[pl.ds(tokbase, CHUNK)], buf)
        handles[c] = pltpu.async_copy(buf, xs_hbm.at[pos2d.at[c]], sem)
    handles[NCHUNK - 2].wait()
    handles[NCHUNK - 1].wait()


def _dispatch_sc(eids, x):
    f = functools.partial(
        pl.kernel,
        out_type=[
            jax.ShapeDtypeStruct((NPAD, MODEL_DIM), jnp.float32),
            jax.ShapeDtypeStruct((N,), jnp.int32),
            jax.ShapeDtypeStruct((NWEXP,), jnp.int32),
            jax.ShapeDtypeStruct((LANES,), jnp.int32),
            jax.ShapeDtypeStruct((LANES,), jnp.int32),
        ],
        mesh=_sc_mesh(),
        compiler_params=pltpu.CompilerParams(needs_layout_passes=False),
        scratch_types=[
            pltpu.VMEM((N,), jnp.int32),
            pltpu.VMEM((LANES,), jnp.int32),
            pltpu.VMEM((LANES,), jnp.int32),
            pltpu.VMEM((NCHUNK, CHUNK), jnp.int32),
            pltpu.VMEM((SLOTS_W,), jnp.int32),
            pltpu.VMEM((LANES,), jnp.int32),
            pltpu.VMEM((NWEXP,), jnp.int32),
            pltpu.VMEM((CHUNK, MODEL_DIM), jnp.float32),
            pltpu.VMEM((CHUNK, MODEL_DIM), jnp.float32),
            pltpu.SemaphoreType.DMA,
            pltpu.SemaphoreType.DMA,
        ],
    )(_dispatch_body)
    return f(eids, x)


def _combine_body(rows_hbm, inv_hbm, out_hbm, idx_v, r1, r2, ob, s1, s2):
    wid = lax.axis_index("c") * NS + lax.axis_index("s")
    tb = wid * TOK_W
    pltpu.sync_copy(inv_hbm.at[pl.ds(tb, CHUNK)], idx_v.at[0])
    pltpu.sync_copy(inv_hbm.at[pl.ds(tb + CHUNK, CHUNK)], idx_v.at[1])
    pltpu.sync_copy(inv_hbm.at[pl.ds(T + tb, CHUNK)], idx_v.at[2])
    pltpu.sync_copy(inv_hbm.at[pl.ds(T + tb + CHUNK, CHUNK)], idx_v.at[3])
    vregs_row = MODEL_DIM // LANES
    for c in range(TOK_W // CHUNK):
        h1 = pltpu.async_copy(rows_hbm.at[idx_v.at[c]], r1, s1)
        h2 = pltpu.async_copy(rows_hbm.at[idx_v.at[2 + c]], r2, s2)
        h1.wait()
        h2.wait()

        def add_row(i, _):
            for j in range(vregs_row):
                sl = pl.ds(j * LANES, LANES)
                ob[i, sl] = r1[i, sl] + r2[i, sl]
            return 0

        lax.fori_loop(0, CHUNK, add_row, 0)
        pltpu.sync_copy(ob, out_hbm.at[pl.ds(tb + c * CHUNK, CHUNK)])


def _combine_sc(rows, inv):
    f = functools.partial(
        pl.kernel,
        out_type=jax.ShapeDtypeStruct((T, MODEL_DIM), jnp.float32),
        mesh=_sc_mesh(),
        compiler_params=pltpu.CompilerParams(needs_layout_passes=False),
        scratch_types=[
            pltpu.VMEM((4, CHUNK), jnp.int32),
            pltpu.VMEM((CHUNK, MODEL_DIM), jnp.float32),
            pltpu.VMEM((CHUNK, MODEL_DIM), jnp.float32),
            pltpu.VMEM((CHUNK, MODEL_DIM), jnp.float32),
            pltpu.SemaphoreType.DMA,
            pltpu.SemaphoreType.DMA,
        ],
    )(_combine_body)
    return f(rows, inv)


def _dispatch_jnp(x, eids):
    counts = jnp.bincount(eids, length=NUM_EXPERTS)
    padded = (counts + BM - 1) // BM * BM
    pad_start = jnp.concatenate([jnp.zeros((1,), jnp.int32),
                                 jnp.cumsum(padded)[:-1].astype(jnp.int32)])
    cs_excl = jnp.concatenate([jnp.zeros((1,), jnp.int32),
                               jnp.cumsum(counts)[:-1].astype(jnp.int32)])
    order = jnp.argsort(eids, stable=True)
    eids_sorted = eids[order]
    rank = jnp.arange(N, dtype=jnp.int32) - cs_excl[eids_sorted]
    pos_sorted = pad_start[eids_sorted] + rank
    inv = jnp.zeros((N,), jnp.int32).at[order].set(pos_sorted)
    tok = jnp.arange(N, dtype=jnp.int32) % T
    x_sorted = jnp.zeros((NPAD, MODEL_DIM), x.dtype).at[inv].set(x[tok])
    pad_incl = pad_start + padded
    wstarts = jnp.arange(NWIN, dtype=jnp.int32) * BM
    wexp = jnp.minimum(
        jnp.sum(wstarts[:, None] >= pad_incl[None, :], axis=1),
        NUM_EXPERTS - 1).astype(jnp.int32)
    return x_sorted, inv, wexp, pad_start.astype(jnp.int32), counts.astype(jnp.int32)


def _gemm_body(wexp_ref, ps_ref, cnt_ref, xs_ref, u_ref, d_ref, out_ref):
    w = pl.program_id(0)
    e = wexp_ref[w]
    loc = (lax.broadcasted_iota(jnp.int32, (BM, 1), 0)
           + w * BM - ps_ref[e])
    active = loc < cnt_ref[e]
    x = jnp.where(active, xs_ref[...], 0.0).astype(jnp.bfloat16)
    u = lax.dot_general(x, u_ref[0], (((1,), (0,)), ((), ())),
                        preferred_element_type=jnp.float32)
    h = (u * u).astype(jnp.bfloat16)
    out_ref[...] = lax.dot_general(h, d_ref[0], (((1,), (0,)), ((), ())),
                                   preferred_element_type=jnp.float32)


def _grouped_gemm(x_sorted, u, d, wexp, pad_start, counts):
    grid_spec = pltpu.PrefetchScalarGridSpec(
        num_scalar_prefetch=3,
        grid=(NWIN,),
        in_specs=[
            pl.BlockSpec((BM, MODEL_DIM), lambda i, wexp, ps, cnt: (i, 0)),
            pl.BlockSpec((1, MODEL_DIM, INTER_DIM),
                         lambda i, wexp, ps, cnt: (wexp[i], 0, 0)),
            pl.BlockSpec((1, INTER_DIM, MODEL_DIM),
                         lambda i, wexp, ps, cnt: (wexp[i], 0, 0)),
        ],
        out_specs=pl.BlockSpec((BM, MODEL_DIM), lambda i, wexp, ps, cnt: (i, 0)),
    )
    return pl.pallas_call(
        _gemm_body,
        grid_spec=grid_spec,
        out_shape=jax.ShapeDtypeStruct((NPAD, MODEL_DIM), jnp.float32),
        interpret=_INTERPRET,
    )(wexp, pad_start, counts, x_sorted, u, d)


@functools.partial(jax.jit, static_argnames=())
def kernel(hidden_states, gate_w, gate_up_w, down_w):
    orig_shape = hidden_states.shape
    x = hidden_states.reshape(-1, orig_shape[-1])
    u = gate_up_w[:, :, INTER_DIM:].astype(jnp.bfloat16)
    d = down_w.astype(jnp.bfloat16)

    e1, e2 = _routing(x, gate_w.T)
    eids = jnp.concatenate([e1, e2], axis=0).reshape(-1)

    x_sorted, inv, wexp, pad_start, counts = _dispatch_sc(eids, x)
    rows = _grouped_gemm(x_sorted, u, d, wexp[:NWIN], pad_start, counts)
    out = _combine_sc(rows, inv)
    return out.reshape(orig_shape)
